# Initial kernel scaffold; baseline (speedup 1.0000x reference)
import jax
import jax.numpy as jnp
from jax.experimental import pallas as pl


def kernel(left_idx, term_idx, right_idx, emb_table, m_w, clf_w, clf_b):
    raise NotImplementedError("write your pallas kernel here")


# speedup vs baseline: 3.3602x; 3.3602x over previous
import functools

import jax
import jax.numpy as jnp
from jax import lax
from jax.experimental import pallas as pl
from jax.experimental.pallas import tpu as pltpu
from jax.experimental.pallas import tpu_sc as plsc

B, L, V, D, O = 16384, 50, 1000000, 64, 5
NC, NS, LANES = 2, 16, 16
NW = NC * NS
SEG = 3 * B
SEGW = SEG // NW
G = 96
NGROUPS = SEGW // G
NBUF = 4
INV_L = 1.0 / L


def _pool_body(idx_hbm, table_hbm, out_hbm, idx_v, rows_v, pooled_v, *sems):
    w = lax.axis_index("s") * NC + lax.axis_index("c")
    seg0 = w * SEGW

    def _issue(slot, s):
        pltpu.async_copy(table_hbm.at[idx_v.at[s]], rows_v.at[slot], sems[slot])

    def _wait(slot):
        pltpu.make_async_copy(
            table_hbm.at[idx_v.at[0]], rows_v.at[slot], sems[slot]
        ).wait()

    def _group(g, carry):
        gbase = seg0 + g * G
        pltpu.sync_copy(idx_hbm.at[pl.ds(gbase, G)], idx_v)
        for b in range(NBUF):
            _issue(b, b)

        def _ring(i, carry):
            s = i * NBUF
            for b in range(NBUF):
                _wait(b)
                for d in range(D // LANES):
                    acc = rows_v[b, 0, pl.ds(d * LANES, LANES)]
                    for r in range(1, L):
                        acc = acc + rows_v[b, r, pl.ds(d * LANES, LANES)]
                    pooled_v[s + b, pl.ds(d * LANES, LANES)] = acc * INV_L
                nxt = s + b + NBUF

                @pl.when(nxt < G)
                def _():
                    _issue(b, nxt)

            return carry

        lax.fori_loop(0, G // NBUF, _ring, 0)
        pltpu.sync_copy(pooled_v, out_hbm.at[pl.ds(gbase, G)])
        return carry

    lax.fori_loop(0, NGROUPS, _group, 0)


def _pool(idx_all, emb_table):
    mesh = plsc.VectorSubcoreMesh(core_axis_name="c", subcore_axis_name="s")
    return pl.kernel(
        _pool_body,
        out_type=jax.ShapeDtypeStruct((SEG, D), jnp.float32),
        mesh=mesh,
        scratch_types=[
            pltpu.VMEM((G, L), jnp.int32),
            pltpu.VMEM((NBUF, L, D), jnp.float32),
            pltpu.VMEM((G, D), jnp.float32),
        ]
        + [pltpu.SemaphoreType.DMA] * NBUF,
        compiler_params=pltpu.CompilerParams(use_tc_tiling_on_sc=False),
    )(idx_all, emb_table)


def _head_body(pooled_ref, mw_ref, clfw_ref, clfb_ref, out_ref):
    mw = mw_ref[...]
    fw = clfw_ref[...]
    logits = clfb_ref[...]
    for wdx in range(3):
        f = jnp.dot(
            fw[:, wdx * D : (wdx + 1) * D], mw, preferred_element_type=jnp.float32
        )
        logits = logits + jnp.dot(
            pooled_ref[wdx], f.T, preferred_element_type=jnp.float32
        )
    m = jnp.max(logits, axis=1, keepdims=True)
    e = jnp.exp(logits - m)
    out_ref[...] = e / jnp.sum(e, axis=1, keepdims=True)


def _head(pooled, m_w, clf_w, clf_b, bm=4096):
    return pl.pallas_call(
        _head_body,
        grid=(B // bm,),
        in_specs=[
            pl.BlockSpec((3, bm, D), lambda i: (0, i, 0)),
            pl.BlockSpec((D, D), lambda i: (0, 0)),
            pl.BlockSpec((O, 3 * D), lambda i: (0, 0)),
            pl.BlockSpec((1, O), lambda i: (0, 0)),
        ],
        out_specs=pl.BlockSpec((bm, O), lambda i: (i, 0)),
        out_shape=jax.ShapeDtypeStruct((B, O), jnp.float32),
    )(pooled, m_w, clf_w, clf_b)


def kernel(left_idx, term_idx, right_idx, emb_table, m_w, clf_w, clf_b):
    idx_all = jnp.concatenate(
        [
            left_idx.astype(jnp.int32),
            term_idx.astype(jnp.int32),
            right_idx.astype(jnp.int32),
        ],
        axis=0,
    )
    pooled = _pool(idx_all, emb_table).reshape(3, B, D)
    return _head(pooled, m_w, clf_w, clf_b.reshape(1, O))


# speedup vs baseline: 4.4545x; 1.3257x over previous
import functools

import jax
import jax.numpy as jnp
from jax import lax
from jax.experimental import pallas as pl
from jax.experimental.pallas import tpu as pltpu
from jax.experimental.pallas import tpu_sc as plsc

B, L, V, D, O = 16384, 50, 1000000, 64, 5
NC, NS, LANES = 2, 16, 16
NW = NC * NS
SEG = 3 * B
SEGW = SEG // NW
G = 96
NBUF = 8
KG = G // NBUF
INV_L = 1.0 / L


def _pool_body(idx_hbm, table_hbm, out_hbm, idx_v, rows_v, pooled_v, *sems):
    w = lax.axis_index("s") * NC + lax.axis_index("c")
    seg0 = w * SEGW

    def _issue(slot, s):
        pltpu.async_copy(table_hbm.at[idx_v.at[s]], rows_v.at[slot], sems[slot])

    def _wait(slot):
        pltpu.make_async_copy(
            table_hbm.at[idx_v.at[0]], rows_v.at[slot], sems[slot]
        ).wait()

    pltpu.sync_copy(idx_hbm.at[pl.ds(seg0, SEGW)], idx_v)
    for b in range(NBUF):
        _issue(b, b)

    def _ring(i, carry):
        s = i * NBUF
        so = lax.rem(s, G)
        for b in range(NBUF):
            _wait(b)
            for d in range(D // LANES):
                sl = pl.ds(d * LANES, LANES)
                a0 = rows_v[b, 0, sl]
                a1 = rows_v[b, 1, sl]
                for r in range(2, L, 2):
                    a0 = a0 + rows_v[b, r, sl]
                    a1 = a1 + rows_v[b, r + 1, sl]
                pooled_v[so + b, sl] = (a0 + a1) * INV_L
            nxt = s + b + NBUF

            @pl.when(nxt < SEGW)
            def _():
                _issue(b, nxt)

        @pl.when(lax.rem(i + 1, KG) == 0)
        def _():
            g0 = (i + 1 - KG) * NBUF
            pltpu.sync_copy(pooled_v, out_hbm.at[pl.ds(seg0 + g0, G)])

        return carry

    lax.fori_loop(0, SEGW // NBUF, _ring, 0)


def _pool(idx_all, emb_table):
    mesh = plsc.VectorSubcoreMesh(core_axis_name="c", subcore_axis_name="s")
    return pl.kernel(
        _pool_body,
        out_type=jax.ShapeDtypeStruct((SEG, D), jnp.float32),
        mesh=mesh,
        scratch_types=[
            pltpu.VMEM((SEGW, L), jnp.int32),
            pltpu.VMEM((NBUF, L, D), jnp.float32),
            pltpu.VMEM((G, D), jnp.float32),
        ]
        + [pltpu.SemaphoreType.DMA] * NBUF,
        compiler_params=pltpu.CompilerParams(use_tc_tiling_on_sc=False),
    )(idx_all, emb_table)


def _head_body(pooled_ref, mw_ref, clfw_ref, clfb_ref, out_ref):
    mw = mw_ref[...]
    fw = clfw_ref[...]
    logits = clfb_ref[...]
    for wdx in range(3):
        f = jnp.dot(
            fw[:, wdx * D : (wdx + 1) * D], mw, preferred_element_type=jnp.float32
        )
        logits = logits + jnp.dot(
            pooled_ref[wdx], f.T, preferred_element_type=jnp.float32
        )
    m = jnp.max(logits, axis=1, keepdims=True)
    e = jnp.exp(logits - m)
    out_ref[...] = e / jnp.sum(e, axis=1, keepdims=True)


def _head(pooled, m_w, clf_w, clf_b, bm=4096):
    return pl.pallas_call(
        _head_body,
        grid=(B // bm,),
        in_specs=[
            pl.BlockSpec((3, bm, D), lambda i: (0, i, 0)),
            pl.BlockSpec((D, D), lambda i: (0, 0)),
            pl.BlockSpec((O, 3 * D), lambda i: (0, 0)),
            pl.BlockSpec((1, O), lambda i: (0, 0)),
        ],
        out_specs=pl.BlockSpec((bm, O), lambda i: (i, 0)),
        out_shape=jax.ShapeDtypeStruct((B, O), jnp.float32),
    )(pooled, m_w, clf_w, clf_b)


def kernel(left_idx, term_idx, right_idx, emb_table, m_w, clf_w, clf_b):
    idx_all = jnp.concatenate(
        [
            left_idx.astype(jnp.int32),
            term_idx.astype(jnp.int32),
            right_idx.astype(jnp.int32),
        ],
        axis=0,
    )
    pooled = _pool(idx_all, emb_table).reshape(3, B, D)
    return _head(pooled, m_w, clf_w, clf_b.reshape(1, O))
